# baseline (speedup 1.0000x reference)
import jax
import jax.numpy as jnp
from jax import lax
from jax.experimental import pallas as pl
from jax.experimental.pallas import tpu as pltpu
from jax.experimental.pallas import tpu_sc as plsc

_B = 2
_C = 64
_HW = 1024
_S = 512
_NC = 2
_NS = 16
_TOK_PER_W = (_B * _HW) // (_NC * _NS)


def _topk_body(z_ref, emb_ref, i1_ref, i2_ref):
    z = z_ref[...].reshape(_B, _C, _HW)
    emb = emb_ref[...]
    en = jnp.sum(emb * emb, axis=1, keepdims=True)
    out1, out2 = [], []
    for b in range(_B):
        zb = z[b]
        xe = lax.dot_general(emb, zb, (((1,), (0,)), ((), ())),
                             precision=lax.Precision.HIGHEST,
                             preferred_element_type=jnp.float32)
        scores = en - 2.0 * xe
        iota = lax.broadcasted_iota(jnp.int32, scores.shape, 0)
        m1 = jnp.min(scores, axis=0, keepdims=True)
        r = scores - m1
        key = (lax.bitcast_convert_type(r, jnp.int32) &
               jnp.int32(~(_S - 1))) | iota
        k1 = jnp.min(key, axis=0, keepdims=True)
        i1 = k1 & (_S - 1)
        masked = jnp.where(key == k1, jnp.int32(0x7FFFFFFF), key)
        k2 = jnp.min(masked, axis=0, keepdims=True)
        i2 = k2 & (_S - 1)
        out1.append(i1)
        out2.append(i2)
    i1_ref[...] = jnp.concatenate(out1, axis=0)
    i2_ref[...] = jnp.concatenate(out2, axis=0)


def _sc_body(emb_hbm, z_hbm, i1_hbm, i2_hbm, q_hbm, st_hbm, io_hbm,
             x_v, r1_v, r2_v, g1_v, g2_v, ds1_v, ds2_v, q_v, st_v,
             i1_v, i2_v, io_v, sem1, sem2, sem3):
    cid = lax.axis_index("c")
    sid = lax.axis_index("s")
    wid = sid * _NC + cid
    b = wid // _NS
    hwb = (wid % _NS) * _TOK_PER_W
    tsl = pl.ds(hwb, _TOK_PER_W)

    pltpu.sync_copy(i1_hbm.at[b, tsl], i1_v)
    pltpu.sync_copy(i2_hbm.at[b, tsl], i2_v)
    cx = pltpu.async_copy(z_hbm.at[b, :, tsl], x_v, sem3)

    zeros16 = jnp.zeros((16,), jnp.int32)
    iota16 = lax.broadcasted_iota(jnp.int32, (16,), 0)
    ngroups = _TOK_PER_W // 16
    idx_vecs = [(i1_v[pl.ds(g * 16, 16)], i2_v[pl.ds(g * 16, 16)])
                for g in range(ngroups)]

    def fire(g):
        sl16 = pl.ds(g * 16, 16)
        i1g, i2g = idx_vecs[g]
        h1 = pltpu.async_copy(emb_hbm.at[i1g], r1_v.at[sl16, :],
                              sem1.at[g % 2])
        h2 = pltpu.async_copy(emb_hbm.at[i2g], r2_v.at[sl16, :],
                              sem2.at[g % 2])
        return h1, h2

    pend = fire(0)
    cx.wait()
    for g in range(ngroups):
        sl = pl.ds(g * 16, 16)
        i1, i2 = idx_vecs[g]
        tok = iota16 + g * 16
        pend[0].wait()
        pend[1].wait()
        if g + 1 < ngroups:
            pend = fire(g + 1)

        @plsc.parallel_loop(0, _C, unroll=4)
        def cbody(c, tok=tok, sl=sl):
            col = zeros16 + c
            e1c = plsc.load_gather(r1_v, [tok, col])
            e2c = plsc.load_gather(r2_v, [tok, col])
            xc = x_v[c, sl]
            g1_v[c, sl] = e1c
            g2_v[c, sl] = e2c
            t1 = xc - e1c
            t2 = xc - e2c
            ds1_v[c, :] = t1 * t1
            ds2_v[c, :] = t2 * t2

        for stride in (32, 16, 8, 4, 2, 1):
            def tbody(i, carry, s=stride):
                ds1_v[i, :] = ds1_v[i, :] + ds1_v[i + s, :]
                ds2_v[i, :] = ds2_v[i, :] + ds2_v[i + s, :]
                return carry

            lax.fori_loop(0, stride, tbody, 0)

        d1 = ds1_v[0, :]
        d2 = ds2_v[0, :]
        take1 = (d1 < d2) | ((d1 == d2) & (i1 < i2))
        io_v[sl] = jnp.where(take1, i1, i2)

        @plsc.parallel_loop(0, _C, unroll=4)
        def qbody(c, take1=take1, sl=sl):
            xc = x_v[c, sl]
            qc = jnp.where(take1, g1_v[c, sl], g2_v[c, sl])
            q_v[c, sl] = qc
            st_v[c, sl] = xc + (qc - xc)

    o1 = pltpu.async_copy(q_v, q_hbm.at[b, :, tsl], sem1.at[0])
    o2 = pltpu.async_copy(st_v, st_hbm.at[b, :, tsl], sem2.at[0])
    o3 = pltpu.async_copy(io_v, io_hbm.at[b, tsl], sem3)
    o1.wait()
    o2.wait()
    o3.wait()


_SC_SCRATCH = [
    pltpu.VMEM((_C, _TOK_PER_W), jnp.float32),
    pltpu.VMEM((_TOK_PER_W, _C), jnp.float32),
    pltpu.VMEM((_TOK_PER_W, _C), jnp.float32),
    pltpu.VMEM((_C, _TOK_PER_W), jnp.float32),
    pltpu.VMEM((_C, _TOK_PER_W), jnp.float32),
    pltpu.VMEM((_C, 16), jnp.float32),
    pltpu.VMEM((_C, 16), jnp.float32),
    pltpu.VMEM((_C, _TOK_PER_W), jnp.float32),
    pltpu.VMEM((_C, _TOK_PER_W), jnp.float32),
    pltpu.VMEM((_TOK_PER_W,), jnp.int32),
    pltpu.VMEM((_TOK_PER_W,), jnp.int32),
    pltpu.VMEM((_TOK_PER_W,), jnp.int32),
    pltpu.SemaphoreType.DMA((2,)),
    pltpu.SemaphoreType.DMA((2,)),
    pltpu.SemaphoreType.DMA,
]

_sc_quantize = pl.kernel(
    _sc_body,
    out_type=[
        jax.ShapeDtypeStruct((_B, _C, _HW), jnp.float32),
        jax.ShapeDtypeStruct((_B, _C, _HW), jnp.float32),
        jax.ShapeDtypeStruct((_B, _HW), jnp.int32),
    ],
    mesh=plsc.VectorSubcoreMesh(core_axis_name="c", subcore_axis_name="s"),
    scratch_types=_SC_SCRATCH,
    compiler_params=pltpu.CompilerParams(use_tc_tiling_on_sc=False,
                                         needs_layout_passes=False),
)


def kernel(z, embeddings):
    b, c, h, w = z.shape
    z3 = z.reshape(b, c, h * w)
    i1, i2 = pl.pallas_call(
        _topk_body,
        out_shape=[
            jax.ShapeDtypeStruct((_B, _HW), jnp.int32),
            jax.ShapeDtypeStruct((_B, _HW), jnp.int32),
        ],
    )(z, embeddings)
    q, st, io = _sc_quantize(embeddings, z3, i1, i2)
    quantized = q.reshape(b, c, h, w)
    st_out = st.reshape(b, c, h, w)
    encoding_indices = io.reshape(b, h, w)
    return (quantized, st_out, encoding_indices)


# speedup vs baseline: 1.0914x; 1.0914x over previous
import jax
import jax.numpy as jnp
from jax import lax
from jax.experimental import pallas as pl
from jax.experimental.pallas import tpu as pltpu
from jax.experimental.pallas import tpu_sc as plsc

_B = 2
_C = 64
_HW = 1024
_S = 512
_NC = 2
_NS = 16
_TOK_PER_W = (_B * _HW) // (_NC * _NS)


def _topk_body(z_ref, emb_ref, i1_ref, i2_ref):
    z = z_ref[...]
    emb = emb_ref[...]
    en = jnp.sum(emb * emb, axis=1, keepdims=True)
    out1, out2 = [], []
    for b in range(_B):
        zb = z[b]
        xe = lax.dot_general(emb, zb, (((1,), (0,)), ((), ())),
                             precision=lax.Precision.HIGHEST,
                             preferred_element_type=jnp.float32)
        scores = en - 2.0 * xe
        iota = lax.broadcasted_iota(jnp.int32, scores.shape, 0)
        m1 = jnp.min(scores, axis=0, keepdims=True)
        r = scores - m1
        key = (lax.bitcast_convert_type(r, jnp.int32) &
               jnp.int32(~(_S - 1))) | iota
        k1 = jnp.min(key, axis=0, keepdims=True)
        i1 = k1 & (_S - 1)
        masked = jnp.where(key == k1, jnp.int32(0x7FFFFFFF), key)
        k2 = jnp.min(masked, axis=0, keepdims=True)
        i2 = k2 & (_S - 1)
        out1.append(i1)
        out2.append(i2)
    i1_ref[...] = jnp.concatenate(out1, axis=0)
    i2_ref[...] = jnp.concatenate(out2, axis=0)


def _sc_body(emb_hbm, z_hbm, i1_hbm, i2_hbm, q_hbm, st_hbm, io_hbm,
             x_v, r1_v, r2_v, g1_v, g2_v, ds1_v, ds2_v, q_v, st_v,
             i1_v, i2_v, io_v, sem1, sem2, sem3):
    cid = lax.axis_index("c")
    sid = lax.axis_index("s")
    wid = sid * _NC + cid
    b = wid // _NS
    hwb = (wid % _NS) * _TOK_PER_W
    tsl = pl.ds(hwb, _TOK_PER_W)

    pltpu.sync_copy(i1_hbm.at[b, tsl], i1_v)
    pltpu.sync_copy(i2_hbm.at[b, tsl], i2_v)
    cx = pltpu.async_copy(z_hbm.at[b, :, tsl], x_v, sem3)

    zeros16 = jnp.zeros((16,), jnp.int32)
    iota16 = lax.broadcasted_iota(jnp.int32, (16,), 0)
    ngroups = _TOK_PER_W // 16
    idx_vecs = [(i1_v[pl.ds(g * 16, 16)], i2_v[pl.ds(g * 16, 16)])
                for g in range(ngroups)]

    def fire(g):
        sl16 = pl.ds(g * 16, 16)
        i1g, i2g = idx_vecs[g]
        h1 = pltpu.async_copy(emb_hbm.at[i1g], r1_v.at[sl16, :],
                              sem1.at[g % 2])
        h2 = pltpu.async_copy(emb_hbm.at[i2g], r2_v.at[sl16, :],
                              sem2.at[g % 2])
        return h1, h2

    pend = fire(0)
    cx.wait()
    for g in range(ngroups):
        sl = pl.ds(g * 16, 16)
        i1, i2 = idx_vecs[g]
        tok = iota16 + g * 16
        pend[0].wait()
        pend[1].wait()
        if g + 1 < ngroups:
            pend = fire(g + 1)

        @plsc.parallel_loop(0, _C, unroll=4)
        def cbody(c, tok=tok, sl=sl):
            col = zeros16 + c
            e1c = plsc.load_gather(r1_v, [tok, col])
            e2c = plsc.load_gather(r2_v, [tok, col])
            xc = x_v[c, sl]
            g1_v[c, sl] = e1c
            g2_v[c, sl] = e2c
            t1 = xc - e1c
            t2 = xc - e2c
            ds1_v[c, :] = t1 * t1
            ds2_v[c, :] = t2 * t2

        for stride in (32, 16, 8, 4, 2, 1):
            def tbody(i, carry, s=stride):
                ds1_v[i, :] = ds1_v[i, :] + ds1_v[i + s, :]
                ds2_v[i, :] = ds2_v[i, :] + ds2_v[i + s, :]
                return carry

            lax.fori_loop(0, stride, tbody, 0)

        d1 = ds1_v[0, :]
        d2 = ds2_v[0, :]
        take1 = (d1 < d2) | ((d1 == d2) & (i1 < i2))
        io_v[sl] = jnp.where(take1, i1, i2)

        @plsc.parallel_loop(0, _C, unroll=4)
        def qbody(c, take1=take1, sl=sl):
            xc = x_v[c, sl]
            qc = jnp.where(take1, g1_v[c, sl], g2_v[c, sl])
            q_v[c, sl] = qc
            st_v[c, sl] = xc + (qc - xc)

    o1 = pltpu.async_copy(q_v, q_hbm.at[b, :, tsl], sem1.at[0])
    o2 = pltpu.async_copy(st_v, st_hbm.at[b, :, tsl], sem2.at[0])
    o3 = pltpu.async_copy(io_v, io_hbm.at[b, tsl], sem3)
    o1.wait()
    o2.wait()
    o3.wait()


_SC_SCRATCH = [
    pltpu.VMEM((_C, _TOK_PER_W), jnp.float32),
    pltpu.VMEM((_TOK_PER_W, _C), jnp.float32),
    pltpu.VMEM((_TOK_PER_W, _C), jnp.float32),
    pltpu.VMEM((_C, _TOK_PER_W), jnp.float32),
    pltpu.VMEM((_C, _TOK_PER_W), jnp.float32),
    pltpu.VMEM((_C, 16), jnp.float32),
    pltpu.VMEM((_C, 16), jnp.float32),
    pltpu.VMEM((_C, _TOK_PER_W), jnp.float32),
    pltpu.VMEM((_C, _TOK_PER_W), jnp.float32),
    pltpu.VMEM((_TOK_PER_W,), jnp.int32),
    pltpu.VMEM((_TOK_PER_W,), jnp.int32),
    pltpu.VMEM((_TOK_PER_W,), jnp.int32),
    pltpu.SemaphoreType.DMA((2,)),
    pltpu.SemaphoreType.DMA((2,)),
    pltpu.SemaphoreType.DMA,
]

_sc_quantize = pl.kernel(
    _sc_body,
    out_type=[
        jax.ShapeDtypeStruct((_B, _C, _HW), jnp.float32),
        jax.ShapeDtypeStruct((_B, _C, _HW), jnp.float32),
        jax.ShapeDtypeStruct((_B, _HW), jnp.int32),
    ],
    mesh=plsc.VectorSubcoreMesh(core_axis_name="c", subcore_axis_name="s"),
    scratch_types=_SC_SCRATCH,
    compiler_params=pltpu.CompilerParams(use_tc_tiling_on_sc=False,
                                         needs_layout_passes=False),
)


def kernel(z, embeddings):
    b, c, h, w = z.shape
    z3 = z.reshape(b, c, h * w)
    i1, i2 = pl.pallas_call(
        _topk_body,
        out_shape=[
            jax.ShapeDtypeStruct((_B, _HW), jnp.int32),
            jax.ShapeDtypeStruct((_B, _HW), jnp.int32),
        ],
    )(z3, embeddings)
    q, st, io = _sc_quantize(embeddings, z3, i1, i2)
    quantized = q.reshape(b, c, h, w)
    st_out = st.reshape(b, c, h, w)
    encoding_indices = io.reshape(b, h, w)
    return (quantized, st_out, encoding_indices)


# speedup vs baseline: 1.1039x; 1.0114x over previous
import jax
import jax.numpy as jnp
from jax import lax
from jax.experimental import pallas as pl
from jax.experimental.pallas import tpu as pltpu
from jax.experimental.pallas import tpu_sc as plsc

_B = 2
_C = 64
_HW = 1024
_S = 512
_NC = 2
_NS = 16
_TOK_PER_W = (_B * _HW) // (_NC * _NS)


def _topk_body(z_ref, emb_ref, i12_ref):
    z = z_ref[...]
    emb = emb_ref[...]
    en = jnp.sum(emb * emb, axis=1, keepdims=True)
    out1, out2 = [], []
    for b in range(_B):
        zb = z[b]
        xe = lax.dot_general(emb, zb, (((1,), (0,)), ((), ())),
                             precision=lax.Precision.HIGHEST,
                             preferred_element_type=jnp.float32)
        scores = en - 2.0 * xe
        iota = lax.broadcasted_iota(jnp.int32, scores.shape, 0)
        m1 = jnp.min(scores, axis=0, keepdims=True)
        r = scores - m1
        key = (lax.bitcast_convert_type(r, jnp.int32) &
               jnp.int32(~(_S - 1))) | iota
        k1 = jnp.min(key, axis=0, keepdims=True)
        i1 = k1 & (_S - 1)
        masked = jnp.where(key == k1, jnp.int32(0x7FFFFFFF), key)
        k2 = jnp.min(masked, axis=0, keepdims=True)
        i2 = k2 & (_S - 1)
        out1.append(i1)
        out2.append(i2)
    i12_ref[...] = jnp.concatenate(out1 + out2, axis=0)


def _sc_body(emb_hbm, z_hbm, i12_hbm, q_hbm, st_hbm, io_hbm,
             x_v, r1_v, r2_v, g1_v, g2_v, ds1_v, ds2_v, q_v, st_v,
             i1_v, i2_v, io_v, sem1, sem2, sem3):
    cid = lax.axis_index("c")
    sid = lax.axis_index("s")
    wid = sid * _NC + cid
    b = wid // _NS
    hwb = (wid % _NS) * _TOK_PER_W
    tsl = pl.ds(hwb, _TOK_PER_W)

    pltpu.sync_copy(i12_hbm.at[b, tsl], i1_v)
    pltpu.sync_copy(i12_hbm.at[b + _B, tsl], i2_v)
    cx = pltpu.async_copy(z_hbm.at[b, :, tsl], x_v, sem3)

    zeros16 = jnp.zeros((16,), jnp.int32)
    iota16 = lax.broadcasted_iota(jnp.int32, (16,), 0)
    ngroups = _TOK_PER_W // 16
    idx_vecs = [(i1_v[pl.ds(g * 16, 16)], i2_v[pl.ds(g * 16, 16)])
                for g in range(ngroups)]

    def fire(g):
        sl16 = pl.ds(g * 16, 16)
        i1g, i2g = idx_vecs[g]
        h1 = pltpu.async_copy(emb_hbm.at[i1g], r1_v.at[sl16, :],
                              sem1.at[g % 2])
        h2 = pltpu.async_copy(emb_hbm.at[i2g], r2_v.at[sl16, :],
                              sem2.at[g % 2])
        return h1, h2

    pend = fire(0)
    cx.wait()
    for g in range(ngroups):
        sl = pl.ds(g * 16, 16)
        i1, i2 = idx_vecs[g]
        tok = iota16 + g * 16
        pend[0].wait()
        pend[1].wait()
        if g + 1 < ngroups:
            pend = fire(g + 1)

        @plsc.parallel_loop(0, _C, unroll=4)
        def cbody(c, tok=tok, sl=sl):
            col = zeros16 + c
            e1c = plsc.load_gather(r1_v, [tok, col])
            e2c = plsc.load_gather(r2_v, [tok, col])
            xc = x_v[c, sl]
            g1_v[c, sl] = e1c
            g2_v[c, sl] = e2c
            t1 = xc - e1c
            t2 = xc - e2c
            ds1_v[c, :] = t1 * t1
            ds2_v[c, :] = t2 * t2

        for stride in (32, 16, 8, 4, 2, 1):
            def tbody(i, carry, s=stride):
                ds1_v[i, :] = ds1_v[i, :] + ds1_v[i + s, :]
                ds2_v[i, :] = ds2_v[i, :] + ds2_v[i + s, :]
                return carry

            lax.fori_loop(0, stride, tbody, 0)

        d1 = ds1_v[0, :]
        d2 = ds2_v[0, :]
        take1 = (d1 < d2) | ((d1 == d2) & (i1 < i2))
        io_v[sl] = jnp.where(take1, i1, i2)

        @plsc.parallel_loop(0, _C, unroll=4)
        def qbody(c, take1=take1, sl=sl):
            xc = x_v[c, sl]
            qc = jnp.where(take1, g1_v[c, sl], g2_v[c, sl])
            q_v[c, sl] = qc
            st_v[c, sl] = xc + (qc - xc)

    o1 = pltpu.async_copy(q_v, q_hbm.at[b, :, tsl], sem1.at[0])
    o2 = pltpu.async_copy(st_v, st_hbm.at[b, :, tsl], sem2.at[0])
    o3 = pltpu.async_copy(io_v, io_hbm.at[b, tsl], sem3)
    o1.wait()
    o2.wait()
    o3.wait()


_SC_SCRATCH = [
    pltpu.VMEM((_C, _TOK_PER_W), jnp.float32),
    pltpu.VMEM((_TOK_PER_W, _C), jnp.float32),
    pltpu.VMEM((_TOK_PER_W, _C), jnp.float32),
    pltpu.VMEM((_C, _TOK_PER_W), jnp.float32),
    pltpu.VMEM((_C, _TOK_PER_W), jnp.float32),
    pltpu.VMEM((_C, 16), jnp.float32),
    pltpu.VMEM((_C, 16), jnp.float32),
    pltpu.VMEM((_C, _TOK_PER_W), jnp.float32),
    pltpu.VMEM((_C, _TOK_PER_W), jnp.float32),
    pltpu.VMEM((_TOK_PER_W,), jnp.int32),
    pltpu.VMEM((_TOK_PER_W,), jnp.int32),
    pltpu.VMEM((_TOK_PER_W,), jnp.int32),
    pltpu.SemaphoreType.DMA((2,)),
    pltpu.SemaphoreType.DMA((2,)),
    pltpu.SemaphoreType.DMA,
]

_sc_quantize = pl.kernel(
    _sc_body,
    out_type=[
        jax.ShapeDtypeStruct((_B, _C, _HW), jnp.float32),
        jax.ShapeDtypeStruct((_B, _C, _HW), jnp.float32),
        jax.ShapeDtypeStruct((_B, _HW), jnp.int32),
    ],
    mesh=plsc.VectorSubcoreMesh(core_axis_name="c", subcore_axis_name="s"),
    scratch_types=_SC_SCRATCH,
    compiler_params=pltpu.CompilerParams(use_tc_tiling_on_sc=False,
                                         needs_layout_passes=False),
)


def kernel(z, embeddings):
    b, c, h, w = z.shape
    z3 = z.reshape(b, c, h * w)
    i12 = pl.pallas_call(
        _topk_body,
        out_shape=jax.ShapeDtypeStruct((2 * _B, _HW), jnp.int32),
    )(z3, embeddings)
    q, st, io = _sc_quantize(embeddings, z3, i12)
    quantized = q.reshape(b, c, h, w)
    st_out = st.reshape(b, c, h, w)
    encoding_indices = io.reshape(b, h, w)
    return (quantized, st_out, encoding_indices)
